# baseline (speedup 1.0000x reference)
import jax
import jax.numpy as jnp
from jax import lax
from jax.experimental import pallas as pl
from jax.experimental.pallas import tpu as pltpu
from jax.experimental.pallas import tpu_sc as plsc

BATCH = 32
SEQ = 1024
EMB = 768
LANES = 16

NUM_CORES = 2
NUM_SUBCORES = 16
NUM_WORKERS = NUM_CORES * NUM_SUBCORES
S_PER_W = SEQ // NUM_WORKERS
VECS_PER_ROW = EMB // LANES
NBUF = 4


def _body(xt_hbm, oidx_hbm, tok_hbm, pos_hbm, out_hbm,
          idx2d, oidx2d, pos_v, rows, gsems, osems):
  wid = lax.axis_index("s") * NUM_CORES + lax.axis_index("c")
  s_base = wid * S_PER_W

  pltpu.sync_copy(xt_hbm.at[pl.ds(s_base, S_PER_W)], idx2d)
  pltpu.sync_copy(oidx_hbm.at[pl.ds(s_base, S_PER_W)], oidx2d)
  pltpu.sync_copy(pos_hbm.at[pl.ds(s_base, S_PER_W)], pos_v)

  def add_pos(s_local, rows_v):
    @plsc.parallel_loop(0, VECS_PER_ROW, 1, unroll=2)
    def _(j):
      sl = pl.ds(j * LANES, LANES)
      pvec = pos_v[s_local, sl]
      for r in range(BATCH):
        rows_v[r, sl] = rows_v[r, sl] + pvec

  def gather(s_local, k):
    return pltpu.make_async_copy(
        tok_hbm.at[idx2d.at[s_local]], rows[k], gsems[k])

  def out_copy(s_local, k):
    return pltpu.make_async_copy(
        rows[k], out_hbm.at[oidx2d.at[s_local]], osems[k])

  gather(0, 0).start()
  gather(1, 1).start()

  def group(g, _):
    for k in range(NBUF):
      s = g * NBUF + k
      kp = (k + 2) % NBUF
      @pl.when(s >= 2)
      def _():
        out_copy(s - 2, kp).wait()
      @pl.when(s + 2 < S_PER_W)
      def _():
        gather(s + 2, kp).start()
      gather(s, k).wait()
      add_pos(s, rows[k])
      out_copy(s, k).start()
    return ()

  lax.fori_loop(0, S_PER_W // NBUF, group, (), unroll=False)

  out_copy(S_PER_W - 2, NBUF - 2).wait()
  out_copy(S_PER_W - 1, NBUF - 1).wait()


@jax.jit
def kernel(x, tok_emb, pos_emb):
  mesh = plsc.VectorSubcoreMesh(
      core_axis_name="c", subcore_axis_name="s",
      num_cores=NUM_CORES, num_subcores=NUM_SUBCORES)
  f = pl.kernel(
      _body,
      out_type=jax.ShapeDtypeStruct((BATCH * SEQ, EMB), jnp.float32),
      mesh=mesh,
      scratch_types=[
          pltpu.VMEM((S_PER_W, BATCH), jnp.int32),
          pltpu.VMEM((S_PER_W, BATCH), jnp.int32),
          pltpu.VMEM((S_PER_W, EMB), jnp.float32),
          [pltpu.VMEM((BATCH, EMB), jnp.float32) for _ in range(NBUF)],
          [pltpu.SemaphoreType.DMA for _ in range(NBUF)],
          [pltpu.SemaphoreType.DMA for _ in range(NBUF)],
      ],
  )
  xt = jnp.swapaxes(x.astype(jnp.int32), 0, 1)
  oidx = (jnp.arange(BATCH, dtype=jnp.int32)[None, :] * SEQ
          + jnp.arange(SEQ, dtype=jnp.int32)[:, None])
  out2d = f(xt, oidx, tok_emb, pos_emb)
  return out2d.reshape(BATCH, SEQ, EMB)


# speedup vs baseline: 1.0859x; 1.0859x over previous
import jax
import jax.numpy as jnp
from jax import lax
from jax.experimental import pallas as pl
from jax.experimental.pallas import tpu as pltpu
from jax.experimental.pallas import tpu_sc as plsc

BATCH = 32
SEQ = 1024
EMB = 768
LANES = 16

NUM_CORES = 2
NUM_SUBCORES = 16
NUM_WORKERS = NUM_CORES * NUM_SUBCORES
S_PER_W = SEQ // NUM_WORKERS
VECS_PER_ROW = EMB // LANES
NBUF = 4


def _body(xt_hbm, oidx_hbm, tok_hbm, pos_hbm, out_hbm,
          idx2d, oidx2d, pos_v, rows, gsems, osems):
  wid = lax.axis_index("s") * NUM_CORES + lax.axis_index("c")
  s_base = wid * S_PER_W

  pltpu.sync_copy(xt_hbm.at[pl.ds(s_base, S_PER_W)], idx2d)
  pltpu.sync_copy(oidx_hbm.at[pl.ds(s_base, S_PER_W)], oidx2d)
  pltpu.sync_copy(pos_hbm.at[pl.ds(s_base, S_PER_W)], pos_v)

  def add_pos(s_local, rows_v):
    @plsc.parallel_loop(0, VECS_PER_ROW, 1)
    def _(j):
      sl = pl.ds(j * LANES, LANES)
      pvec = pos_v[s_local, sl]
      for r in range(BATCH):
        rows_v[r, sl] = rows_v[r, sl] + pvec

  def gather(s_local, k):
    return pltpu.make_async_copy(
        tok_hbm.at[idx2d.at[s_local]], rows[k], gsems[k])

  def out_copy(s_local, k):
    return pltpu.make_async_copy(
        rows[k], out_hbm.at[oidx2d.at[s_local]], osems[k])

  gather(0, 0).start()
  gather(1, 1).start()

  def group(g, _):
    for k in range(NBUF):
      s = g * NBUF + k
      kp = (k + 2) % NBUF
      @pl.when(s >= 2)
      def _():
        out_copy(s - 2, kp).wait()
      @pl.when(s + 2 < S_PER_W)
      def _():
        gather(s + 2, kp).start()
      gather(s, k).wait()
      add_pos(s, rows[k])
      out_copy(s, k).start()
    return ()

  lax.fori_loop(0, S_PER_W // NBUF, group, (), unroll=False)

  out_copy(S_PER_W - 2, NBUF - 2).wait()
  out_copy(S_PER_W - 1, NBUF - 1).wait()


@jax.jit
def kernel(x, tok_emb, pos_emb):
  mesh = plsc.VectorSubcoreMesh(
      core_axis_name="c", subcore_axis_name="s",
      num_cores=NUM_CORES, num_subcores=NUM_SUBCORES)
  f = pl.kernel(
      _body,
      out_type=jax.ShapeDtypeStruct((BATCH * SEQ, EMB), jnp.float32),
      mesh=mesh,
      scratch_types=[
          pltpu.VMEM((S_PER_W, BATCH), jnp.int32),
          pltpu.VMEM((S_PER_W, BATCH), jnp.int32),
          pltpu.VMEM((S_PER_W, EMB), jnp.float32),
          [pltpu.VMEM((BATCH, EMB), jnp.float32) for _ in range(NBUF)],
          [pltpu.SemaphoreType.DMA for _ in range(NBUF)],
          [pltpu.SemaphoreType.DMA for _ in range(NBUF)],
      ],
  )
  xt = jnp.swapaxes(x.astype(jnp.int32), 0, 1)
  oidx = (jnp.arange(BATCH, dtype=jnp.int32)[None, :] * SEQ
          + jnp.arange(SEQ, dtype=jnp.int32)[:, None])
  out2d = f(xt, oidx, tok_emb, pos_emb)
  return out2d.reshape(BATCH, SEQ, EMB)


# speedup vs baseline: 1.1088x; 1.0211x over previous
import jax
import jax.numpy as jnp
from jax import lax
from jax.experimental import pallas as pl
from jax.experimental.pallas import tpu as pltpu
from jax.experimental.pallas import tpu_sc as plsc

BATCH = 32
SEQ = 1024
EMB = 768
LANES = 16

NUM_CORES = 2
NUM_SUBCORES = 16
NUM_WORKERS = NUM_CORES * NUM_SUBCORES
S_PER_W = SEQ // NUM_WORKERS
VECS_PER_ROW = EMB // LANES
NBUF = 4


def _body(xt_hbm, oidx_hbm, tok_hbm, pos_hbm, out_hbm,
          idx2d, oidx2d, pos_v, rows, gsems, osems):
  wid = lax.axis_index("s") * NUM_CORES + lax.axis_index("c")
  s_base = wid * S_PER_W

  pltpu.sync_copy(xt_hbm.at[pl.ds(s_base, S_PER_W)], idx2d)

  def add_pos(s_local, rows_v):
    @plsc.parallel_loop(0, VECS_PER_ROW, 1)
    def _(j):
      sl = pl.ds(j * LANES, LANES)
      pvec = pos_v[s_local, sl]
      for r in range(BATCH):
        rows_v[r, sl] = rows_v[r, sl] + pvec

  def gather(s_local, k):
    return pltpu.make_async_copy(
        tok_hbm.at[idx2d.at[s_local]], rows[k], gsems[k])

  def out_copy(s_local, k):
    return pltpu.make_async_copy(
        rows[k], out_hbm.at[oidx2d.at[s_local]], osems[k])

  gather(0, 0).start()
  gather(1, 1).start()
  pltpu.sync_copy(oidx_hbm.at[pl.ds(s_base, S_PER_W)], oidx2d)
  pltpu.sync_copy(pos_hbm.at[pl.ds(s_base, S_PER_W)], pos_v)

  def group(g, _):
    for k in range(NBUF):
      s = g * NBUF + k
      kp = (k + 2) % NBUF
      @pl.when(s >= 2)
      def _():
        out_copy(s - 2, kp).wait()
      @pl.when(s + 2 < S_PER_W)
      def _():
        gather(s + 2, kp).start()
      gather(s, k).wait()
      add_pos(s, rows[k])
      out_copy(s, k).start()
    return ()

  lax.fori_loop(0, S_PER_W // NBUF, group, (), unroll=False)

  out_copy(S_PER_W - 2, NBUF - 2).wait()
  out_copy(S_PER_W - 1, NBUF - 1).wait()


@jax.jit
def kernel(x, tok_emb, pos_emb):
  mesh = plsc.VectorSubcoreMesh(
      core_axis_name="c", subcore_axis_name="s",
      num_cores=NUM_CORES, num_subcores=NUM_SUBCORES)
  f = pl.kernel(
      _body,
      out_type=jax.ShapeDtypeStruct((BATCH * SEQ, EMB), jnp.float32),
      mesh=mesh,
      scratch_types=[
          pltpu.VMEM((S_PER_W, BATCH), jnp.int32),
          pltpu.VMEM((S_PER_W, BATCH), jnp.int32),
          pltpu.VMEM((S_PER_W, EMB), jnp.float32),
          [pltpu.VMEM((BATCH, EMB), jnp.float32) for _ in range(NBUF)],
          [pltpu.SemaphoreType.DMA for _ in range(NBUF)],
          [pltpu.SemaphoreType.DMA for _ in range(NBUF)],
      ],
  )
  xt = jnp.swapaxes(x.astype(jnp.int32), 0, 1)
  oidx = (jnp.arange(BATCH, dtype=jnp.int32)[None, :] * SEQ
          + jnp.arange(SEQ, dtype=jnp.int32)[:, None])
  out2d = f(xt, oidx, tok_emb, pos_emb)
  return out2d.reshape(BATCH, SEQ, EMB)
